# baseline (speedup 1.0000x reference)
import functools

import jax
import jax.numpy as jnp
from jax import lax
from jax.experimental import pallas as pl
from jax.experimental.pallas import tpu as pltpu
from jax.experimental.pallas import tpu_sc as plsc

NUM_EMBEDDINGS = 256
EMBEDDING_DIM = 256

_info = plsc.get_sparse_core_info()
_NC, _NS = _info.num_cores, _info.num_subcores
_NW = _NC * _NS

_B = 1024 * 200
_BPW = _B // _NW
_A = 96
_G = 32
_SS = _A + _G
_STEPS = _BPW // _SS
_TV_ROWS = NUM_EMBEDDINGS * EMBEDDING_DIM // 128


def _make_kernel():
  mesh = plsc.VectorSubcoreMesh(core_axis_name="c", subcore_axis_name="s")

  @functools.partial(
      pl.kernel,
      mesh=mesh,
      out_type=jax.ShapeDtypeStruct((_B, EMBEDDING_DIM), jnp.float32),
      scratch_types=[
          pltpu.VMEM((_TV_ROWS, 128), jnp.float32),
          pltpu.VMEM((_BPW,), jnp.int32),
          pltpu.VMEM((_A, EMBEDDING_DIM), jnp.float32),
          pltpu.VMEM((_A, EMBEDDING_DIM), jnp.float32),
          pltpu.VMEM((_G, EMBEDDING_DIM), jnp.float32),
          pltpu.SemaphoreType.DMA,
          pltpu.SemaphoreType.DMA,
          pltpu.SemaphoreType.DMA,
          pltpu.SemaphoreType.DMA,
          pltpu.SemaphoreType.DMA,
      ],
  )
  def k(idx_hbm, table_hbm, table2d_hbm, out_hbm,
        table_v, idx_v, abuf0, abuf1, gbuf,
        asem0, asem1, gsem, gosem, tsem):
    wid = lax.axis_index("s") * _NC + lax.axis_index("c")
    base = wid * _BPW
    pltpu.async_copy(table2d_hbm, table_v, tsem)
    pltpu.sync_copy(idx_hbm.at[pl.ds(base, _BPW)], idx_v)
    pltpu.make_async_copy(table2d_hbm, table_v, tsem).wait()

    abufs = (abuf0, abuf1)
    asems = (asem0, asem1)
    rows_per_vrow = EMBEDDING_DIM // 128

    def _a_out(s):
      return out_hbm.at[pl.ds(base + s * _SS, _A)]

    def _g_out(s):
      return out_hbm.at[pl.ds(base + s * _SS + _A, _G)]

    def _gather(s):
      pltpu.async_copy(
          table_hbm.at[idx_v.at[pl.ds(s * _SS + _A, _G)]], gbuf, gsem)

    def body(q, _):
      for b in range(2):
        s = 2 * q + b
        off = s * _SS

        @pl.when(s >= 1)
        def _():
          pltpu.make_async_copy(gbuf, _g_out(s - 1), gosem).wait()

        _gather(s)

        @pl.when(s >= 2)
        def _():
          pltpu.make_async_copy(abufs[b], _a_out(s - 2), asems[b]).wait()

        abuf = abufs[b]

        @plsc.parallel_loop(0, _A // 16, unroll=2)
        def group_body(g):
          v16 = idx_v[pl.ds(off + g * 16, 16)]
          vb = v16 * rows_per_vrow
          rbs = [vb[l] for l in range(16)]
          for l in range(16):
            row = g * 16 + l
            vals = [table_v[rbs[l] + h, pl.ds(k8 * 16, 16)]
                    for h in range(rows_per_vrow)
                    for k8 in range(128 // 16)]
            for kk in range(EMBEDDING_DIM // 16):
              abuf[row, pl.ds(kk * 16, 16)] = vals[kk]

        pltpu.async_copy(abuf, _a_out(s), asems[b])
        pltpu.make_async_copy(
            table_hbm.at[idx_v.at[pl.ds(off + _A, _G)]], gbuf, gsem).wait()
        pltpu.async_copy(gbuf, _g_out(s), gosem)
      return 0

    lax.fori_loop(0, _STEPS // 2, body, 0)

    pltpu.make_async_copy(abuf0, _a_out(_STEPS - 2), asem0).wait()
    pltpu.make_async_copy(abuf1, _a_out(_STEPS - 1), asem1).wait()
    pltpu.make_async_copy(gbuf, _g_out(_STEPS - 1), gosem).wait()

  return k


_kernel = _make_kernel()


@jax.jit
def kernel(text_inputs, embedding_table):
  idx = text_inputs.reshape(-1).astype(jnp.int32)
  out = _kernel(idx, embedding_table,
                embedding_table.reshape(_TV_ROWS, 128))
  return out.reshape(text_inputs.shape + (EMBEDDING_DIM,))


# speedup vs baseline: 1.1281x; 1.1281x over previous
import functools

import jax
import jax.numpy as jnp
from jax import lax
from jax.experimental import pallas as pl
from jax.experimental.pallas import tpu as pltpu
from jax.experimental.pallas import tpu_sc as plsc

NUM_EMBEDDINGS = 256
EMBEDDING_DIM = 256

_info = plsc.get_sparse_core_info()
_NC, _NS = _info.num_cores, _info.num_subcores
_NW = _NC * _NS

_B = 1024 * 200
_BPW = _B // _NW
_A = 96
_G = 32
_SS = _A + _G
_STEPS = _BPW // _SS
_TV_ROWS = NUM_EMBEDDINGS * EMBEDDING_DIM // 128


def _make_kernel():
  mesh = plsc.VectorSubcoreMesh(core_axis_name="c", subcore_axis_name="s")

  @functools.partial(
      pl.kernel,
      mesh=mesh,
      out_type=jax.ShapeDtypeStruct((_B, EMBEDDING_DIM), jnp.float32),
      scratch_types=[
          pltpu.VMEM((_TV_ROWS, 128), jnp.float32),
          pltpu.VMEM((_BPW,), jnp.int32),
          pltpu.VMEM((_A, EMBEDDING_DIM), jnp.float32),
          pltpu.VMEM((_A, EMBEDDING_DIM), jnp.float32),
          pltpu.VMEM((_G, EMBEDDING_DIM), jnp.float32),
          pltpu.SemaphoreType.DMA,
          pltpu.SemaphoreType.DMA,
          pltpu.SemaphoreType.DMA,
          pltpu.SemaphoreType.DMA,
          pltpu.SemaphoreType.DMA,
      ],
  )
  def k(idx_hbm, table_hbm, table2d_hbm, out_hbm,
        table_v, idx_v, abuf0, abuf1, gbuf,
        asem0, asem1, gsem, gosem, tsem):
    wid = lax.axis_index("s") * _NC + lax.axis_index("c")
    base = wid * _BPW
    pltpu.async_copy(table2d_hbm, table_v, tsem)
    pltpu.sync_copy(idx_hbm.at[pl.ds(base, _BPW)], idx_v)
    pltpu.make_async_copy(table2d_hbm, table_v, tsem).wait()

    abufs = (abuf0, abuf1)
    asems = (asem0, asem1)
    rows_per_vrow = EMBEDDING_DIM // 128

    def _a_out(s):
      return out_hbm.at[pl.ds(base + s * _SS, _A)]

    def _g_out(s):
      return out_hbm.at[pl.ds(base + s * _SS + _A, _G)]

    def _gather(s):
      pltpu.async_copy(
          table_hbm.at[idx_v.at[pl.ds(s * _SS + _A, _G)]], gbuf, gsem)

    def body(q, _):
      for b in range(2):
        s = 2 * q + b
        off = s * _SS

        @pl.when(s >= 1)
        def _():
          pltpu.make_async_copy(gbuf, _g_out(s - 1), gosem).wait()

        _gather(s)

        @pl.when(s >= 2)
        def _():
          pltpu.make_async_copy(abufs[b], _a_out(s - 2), asems[b]).wait()

        abuf = abufs[b]

        @plsc.parallel_loop(0, _A // 16)
        def group_body(g):
          v16 = idx_v[pl.ds(off + g * 16, 16)]
          vb = v16 * rows_per_vrow
          rbs = [vb[l] for l in range(16)]
          for l in range(16):
            row = g * 16 + l
            vals = [table_v[rbs[l] + h, pl.ds(k8 * 16, 16)]
                    for h in range(rows_per_vrow)
                    for k8 in range(128 // 16)]
            for kk in range(EMBEDDING_DIM // 16):
              abuf[row, pl.ds(kk * 16, 16)] = vals[kk]

        pltpu.async_copy(abuf, _a_out(s), asems[b])
        pltpu.make_async_copy(
            table_hbm.at[idx_v.at[pl.ds(off + _A, _G)]], gbuf, gsem).wait()
        pltpu.async_copy(gbuf, _g_out(s), gosem)
      return 0

    lax.fori_loop(0, _STEPS // 2, body, 0)

    pltpu.make_async_copy(abuf0, _a_out(_STEPS - 2), asem0).wait()
    pltpu.make_async_copy(abuf1, _a_out(_STEPS - 1), asem1).wait()
    pltpu.make_async_copy(gbuf, _g_out(_STEPS - 1), gosem).wait()

  return k


_kernel = _make_kernel()


@jax.jit
def kernel(text_inputs, embedding_table):
  idx = text_inputs.reshape(-1).astype(jnp.int32)
  out = _kernel(idx, embedding_table,
                embedding_table.reshape(_TV_ROWS, 128))
  return out.reshape(text_inputs.shape + (EMBEDDING_DIM,))
